# baseline (speedup 1.0000x reference)
import functools

import jax
import jax.numpy as jnp
import numpy as np
from jax import lax
from jax.experimental import pallas as pl
from jax.experimental.pallas import tpu as pltpu
from jax.experimental.pallas import tpu_sc as plsc

_NUM_CLASSES = 15
_N_PAIRS = 60
_N_LOCAL = 20
_N_MID = 20
_N_GLOBAL = _N_PAIRS - _N_LOCAL - _N_MID
_WINDOW = 5
_NUM_BLOCKS = 4
_D = 200
_BLOCK_SIZE = _D // _NUM_BLOCKS

_NC = 2
_NS = 16
_NW = _NC * _NS
_GW = 128


@functools.lru_cache(maxsize=4)
def _gather_indices(batch: int) -> np.ndarray:
    cpu = jax.local_devices(backend="cpu")[0]
    with jax.default_device(cpu):
        return np.asarray(_gather_indices_eval(batch))


def _gather_indices_eval(batch: int) -> jax.Array:
    key = jax.random.key(42)
    kc, ko, kb, ki, kj, kg1, kg2 = jax.random.split(key, 7)
    centers = jax.random.randint(kc, (batch, _N_LOCAL), _WINDOW, _D - _WINDOW)
    offs = jax.random.randint(ko, (batch, _N_LOCAL), 0, 2 * _WINDOW + 1) - _WINDOW
    offs = jnp.where(offs == 0, 1, offs)
    local = jnp.stack([centers, centers + offs], axis=-1)
    sb = jax.random.randint(kb, (batch, _N_MID), 0, _NUM_BLOCKS - 1)
    i = sb * _BLOCK_SIZE + jax.random.randint(ki, (batch, _N_MID), 0, _BLOCK_SIZE)
    j = (sb + 1) * _BLOCK_SIZE + jax.random.randint(kj, (batch, _N_MID), 0, _BLOCK_SIZE)
    mid = jnp.stack([i, j], axis=-1)
    gi = jax.random.randint(kg1, (batch, _N_GLOBAL), 0, _D)
    gj = jax.random.randint(kg2, (batch, _N_GLOBAL), 0, _D)
    glob = jnp.stack([gi, gj], axis=-1)
    pairs = jnp.concatenate([local, mid, glob], axis=1)

    rows_w = batch // _NW
    rloc = ((jnp.arange(batch, dtype=jnp.int32) % rows_w) * _D)[:, None]
    ci = pairs[:, :, 0].astype(jnp.int32) + rloc
    cj = pairs[:, :, 1].astype(jnp.int32) + rloc
    pad = jnp.broadcast_to(rloc, (batch, 4))
    idx = jnp.concatenate([ci, pad, cj, pad], axis=1)
    return idx.reshape(-1)


def _sc_gather(x_flat: jax.Array, idx_flat: jax.Array, batch: int) -> jax.Array:
    rows_w = batch // _NW
    xw = rows_w * _D
    gw = rows_w * _GW
    mesh = plsc.VectorSubcoreMesh(
        core_axis_name="c", subcore_axis_name="s",
        num_cores=_NC, num_subcores=_NS)

    @functools.partial(
        pl.kernel,
        out_type=jax.ShapeDtypeStruct((batch * _GW,), jnp.float32),
        mesh=mesh,
        compiler_params=pltpu.CompilerParams(needs_layout_passes=False),
        scratch_types=[
            pltpu.VMEM((xw,), jnp.float32),
            pltpu.VMEM((gw,), jnp.int32),
            pltpu.VMEM((gw,), jnp.float32),
        ],
    )
    def sc_kernel(x_hbm, idx_hbm, out_hbm, x_v, idx_v, g_v):
        wid = lax.axis_index("s") * _NC + lax.axis_index("c")
        pltpu.sync_copy(x_hbm.at[pl.ds(wid * xw, xw)], x_v)
        pltpu.sync_copy(idx_hbm.at[pl.ds(wid * gw, gw)], idx_v)

        @plsc.parallel_loop(0, gw, 16, unroll=8)
        def _(off):
            iv = idx_v[pl.ds(off, 16)]
            g_v[pl.ds(off, 16)] = plsc.load_gather(x_v, [iv])

        pltpu.sync_copy(g_v, out_hbm.at[pl.ds(wid * gw, gw)])

    return sc_kernel(x_flat, idx_flat)


def _perm_matrix():
    p_mat = np.zeros((240, 192), np.float32)
    for pair in range(_N_PAIRS):
        p_mat[2 * pair, pair] = 1.0
        p_mat[2 * pair + 1, 64 + pair] = 1.0
        p_mat[120 + 2 * pair, 128 + pair] = 1.0
    return p_mat


_PERM = _perm_matrix()

_gather_indices(4096)


def _tc_body(g_ref, p_ref, ab_ref, o_ref, cs_ref):
    c = pl.program_id(0)

    @pl.when(c == 0)
    def _():
        gt = g_ref[...].T
        cg = jnp.cos(gt)
        sg = jnp.sin(gt)
        v = sg[0:64, :] * sg[64:128, :]
        mt = jnp.concatenate([cg, v], axis=0)
        cs_ref[...] = jnp.dot(p_ref[...], mt, preferred_element_type=jnp.float32)

    o_ref[...] = (0.5 + ab_ref[c, 0] * cs_ref[0:120, :]
                  - ab_ref[c, 1] * cs_ref[120:240, :])


def _tc_expand(g2d: jax.Array, ab: jax.Array) -> jax.Array:
    batch = g2d.shape[0]
    return pl.pallas_call(
        _tc_body,
        grid=(_NUM_CLASSES,),
        in_specs=[
            pl.BlockSpec((batch, _GW), lambda c: (0, 0)),
            pl.BlockSpec((240, 192), lambda c: (0, 0)),
            pl.BlockSpec(memory_space=pltpu.SMEM),
        ],
        out_specs=pl.BlockSpec((120, batch), lambda c: (c, 0)),
        out_shape=jax.ShapeDtypeStruct((1800, batch), jnp.float32),
        scratch_shapes=[pltpu.VMEM((240, batch), jnp.float32)],
    )(g2d, jnp.asarray(_PERM), ab)


def kernel(x, q_params):
    batch = x.shape[0]
    idx = _gather_indices(batch)
    g = _sc_gather(x.reshape(-1), idx, batch)
    a_cls = jnp.cos(q_params[:, 1]) * 0.5
    b_cls = jnp.cos(q_params[:, 0]) * jnp.sin(q_params[:, 1]) * 0.5
    ab = jnp.stack([a_cls, b_cls], axis=1)
    out_t = _tc_expand(g.reshape(batch, _GW), ab)
    return out_t.reshape(_NUM_CLASSES, 2 * _N_PAIRS, batch).transpose(2, 1, 0)


# speedup vs baseline: 1.0612x; 1.0612x over previous
import functools

import jax
import jax.numpy as jnp
import numpy as np
from jax import lax
from jax.experimental import pallas as pl
from jax.experimental.pallas import tpu as pltpu
from jax.experimental.pallas import tpu_sc as plsc

_NUM_CLASSES = 15
_N_PAIRS = 60
_N_LOCAL = 20
_N_MID = 20
_N_GLOBAL = _N_PAIRS - _N_LOCAL - _N_MID
_WINDOW = 5
_NUM_BLOCKS = 4
_D = 200
_BLOCK_SIZE = _D // _NUM_BLOCKS

_NC = 2
_NS = 16
_NW = _NC * _NS
_GW = 128


@functools.lru_cache(maxsize=4)
def _gather_indices(batch: int) -> np.ndarray:
    cpu = jax.local_devices(backend="cpu")[0]
    with jax.default_device(cpu):
        return np.asarray(_gather_indices_eval(batch))


def _gather_indices_eval(batch: int) -> jax.Array:
    key = jax.random.key(42)
    kc, ko, kb, ki, kj, kg1, kg2 = jax.random.split(key, 7)
    centers = jax.random.randint(kc, (batch, _N_LOCAL), _WINDOW, _D - _WINDOW)
    offs = jax.random.randint(ko, (batch, _N_LOCAL), 0, 2 * _WINDOW + 1) - _WINDOW
    offs = jnp.where(offs == 0, 1, offs)
    local = jnp.stack([centers, centers + offs], axis=-1)
    sb = jax.random.randint(kb, (batch, _N_MID), 0, _NUM_BLOCKS - 1)
    i = sb * _BLOCK_SIZE + jax.random.randint(ki, (batch, _N_MID), 0, _BLOCK_SIZE)
    j = (sb + 1) * _BLOCK_SIZE + jax.random.randint(kj, (batch, _N_MID), 0, _BLOCK_SIZE)
    mid = jnp.stack([i, j], axis=-1)
    gi = jax.random.randint(kg1, (batch, _N_GLOBAL), 0, _D)
    gj = jax.random.randint(kg2, (batch, _N_GLOBAL), 0, _D)
    glob = jnp.stack([gi, gj], axis=-1)
    pairs = jnp.concatenate([local, mid, glob], axis=1)

    rows_w = batch // _NW
    lb = (jnp.arange(batch, dtype=jnp.int32) % rows_w)[:, None]
    ci = (pairs[:, :, 0].astype(jnp.int32) << 8) | lb
    cj = (pairs[:, :, 1].astype(jnp.int32) << 8) | lb
    pad = jnp.broadcast_to(lb, (batch, 4))
    idx = jnp.concatenate([ci, pad, cj, pad], axis=1)
    return idx.reshape(-1)


def _sc_gather(x_t: jax.Array, idx_flat: jax.Array, batch: int) -> jax.Array:
    rows_w = batch // _NW
    gw = rows_w * _GW
    mesh = plsc.VectorSubcoreMesh(
        core_axis_name="c", subcore_axis_name="s",
        num_cores=_NC, num_subcores=_NS)

    @functools.partial(
        pl.kernel,
        out_type=jax.ShapeDtypeStruct((batch * _GW,), jnp.float32),
        mesh=mesh,
        compiler_params=pltpu.CompilerParams(needs_layout_passes=False),
        scratch_types=[
            pltpu.VMEM((_D, rows_w), jnp.float32),
            pltpu.VMEM((gw,), jnp.int32),
            pltpu.VMEM((gw,), jnp.float32),
        ],
    )
    def sc_kernel(x_hbm, idx_hbm, out_hbm, x_v, idx_v, g_v):
        wid = lax.axis_index("s") * _NC + lax.axis_index("c")
        pltpu.sync_copy(x_hbm.at[:, pl.ds(wid * rows_w, rows_w)], x_v)
        pltpu.sync_copy(idx_hbm.at[pl.ds(wid * gw, gw)], idx_v)

        @plsc.parallel_loop(0, gw, 16, unroll=8)
        def _(off):
            iv = idx_v[pl.ds(off, 16)]
            rows = jax.lax.shift_right_logical(iv, 8)
            cols = jax.lax.bitwise_and(iv, 255)
            g_v[pl.ds(off, 16)] = plsc.load_gather(x_v, [rows, cols])

        pltpu.sync_copy(g_v, out_hbm.at[pl.ds(wid * gw, gw)])

    return sc_kernel(x_t, idx_flat)


def _perm_matrix():
    p_mat = np.zeros((240, 192), np.float32)
    for pair in range(_N_PAIRS):
        p_mat[2 * pair, pair] = 1.0
        p_mat[2 * pair + 1, 64 + pair] = 1.0
        p_mat[120 + 2 * pair, 128 + pair] = 1.0
    return p_mat


_PERM = _perm_matrix()

_gather_indices(4096)


def _tc_body(g_ref, p_ref, ab_ref, o_ref, cs_ref):
    c = pl.program_id(0)

    @pl.when(c == 0)
    def _():
        gt = g_ref[...].T
        cg = jnp.cos(gt)
        sg = jnp.sin(gt)
        v = sg[0:64, :] * sg[64:128, :]
        mt = jnp.concatenate([cg, v], axis=0)
        cs_ref[...] = jnp.dot(p_ref[...], mt, preferred_element_type=jnp.float32)

    o_ref[...] = (0.5 + ab_ref[c, 0] * cs_ref[0:120, :]
                  - ab_ref[c, 1] * cs_ref[120:240, :])


def _tc_expand(g2d: jax.Array, ab: jax.Array) -> jax.Array:
    batch = g2d.shape[0]
    return pl.pallas_call(
        _tc_body,
        grid=(_NUM_CLASSES,),
        in_specs=[
            pl.BlockSpec((batch, _GW), lambda c: (0, 0)),
            pl.BlockSpec((240, 192), lambda c: (0, 0)),
            pl.BlockSpec(memory_space=pltpu.SMEM),
        ],
        out_specs=pl.BlockSpec((120, batch), lambda c: (c, 0)),
        out_shape=jax.ShapeDtypeStruct((1800, batch), jnp.float32),
        scratch_shapes=[pltpu.VMEM((240, batch), jnp.float32)],
    )(g2d, jnp.asarray(_PERM), ab)


def kernel(x, q_params):
    batch = x.shape[0]
    idx = _gather_indices(batch)
    g = _sc_gather(x.T, idx, batch)
    a_cls = jnp.cos(q_params[:, 1]) * 0.5
    b_cls = jnp.cos(q_params[:, 0]) * jnp.sin(q_params[:, 1]) * 0.5
    ab = jnp.stack([a_cls, b_cls], axis=1)
    out_t = _tc_expand(g.reshape(batch, _GW), ab)
    return out_t.reshape(_NUM_CLASSES, 2 * _N_PAIRS, batch).transpose(2, 1, 0)
